# baseline (speedup 1.0000x reference)
import functools

import jax
import jax.numpy as jnp
from jax import lax
from jax.experimental import pallas as pl
from jax.experimental.pallas import tpu as pltpu
from jax.experimental.pallas import tpu_sc as plsc

N = 10000
E = 320000
H = 128
NF6 = 6
OUT_D = 264

NC = 2
NS = 16
NW = NC * NS
L = 16
TW = 128
NT = E // TW
NJT = OUT_D // 8

CB = 10
NCH = NT // CB
CE = CB * TW

FB = 3
FCH = 26
FT_BASE = 78


def _rsqrt(x):
    xi = lax.bitcast_convert_type(x, jnp.int32)
    yi = jnp.int32(0x5F3759DF) - lax.shift_right_logical(xi, 1)
    y = lax.bitcast_convert_type(yi, jnp.float32)
    xh = x * 0.5
    for _ in range(3):
        y = y * (1.5 - xh * y * y)
    return y


def _edge_body(rc_hbm, weight_hbm, embT_hbm, feat6T_hbm, out_hbm,
               buf, ib0, ib1, wb0, wb1, t0, t1, t2,
               st0, st1, st2, si0, si1):
    wid = lax.axis_index("s") * NC + lax.axis_index("c")
    tiles = (t0, t1, t2)
    tsems = (st0, st1, st2)

    def drain_tile(slot):
        pltpu.make_async_copy(out_hbm.at[0, 0], tiles[slot], tsems[slot]).wait()

    pltpu.sync_copy(embT_hbm.at[pl.ds((wid % 16) * 8 * N, 8 * N)], buf)
    idx_base = (wid // 16) * E

    def e_prefetch(c, ib, sem):
        b = jnp.minimum(c, NCH - 1) * CE
        pltpu.async_copy(rc_hbm.at[pl.ds(idx_base + b, CE)], ib, sem)

    def e_drain(ib, sem):
        pltpu.make_async_copy(rc_hbm.at[pl.ds(0, CE)], ib, sem).wait()

    def e_tile(c, k, slot, drain):
        if drain:
            drain_tile(slot)
        tile = tiles[slot]
        ib = (ib0, ib1)[slot]

    def e_chunk(c, ib, sem, ib_n, sem_n, first):
        e_drain(ib, sem)
        e_prefetch(c + 1, ib_n, sem_n)

        def one_tile(k, slot, drain):
            if drain:
                drain_tile(slot)
            tile = tiles[slot]

            def grp(g, carry):
                iv = ib[pl.ds(k * TW + g * L, L)]
                for jj in range(8):
                    v = plsc.load_gather(buf, [iv + jj * N])
                    tile[pl.ds(jj * TW + g * L, L)] = v
                return carry

            lax.fori_loop(0, 8, grp, 0)
            pltpu.async_copy(tile, out_hbm.at[wid, c * CB + k], tsems[slot])

        def tpair(q, carry):
            one_tile(2 * q, 0, drain=True)
            one_tile(2 * q + 1, 1, drain=True)
            return carry

        if first:
            one_tile(0, 0, drain=False)
            one_tile(1, 1, drain=False)
            lax.fori_loop(1, CB // 2, tpair, 0)
        else:
            lax.fori_loop(0, CB // 2, tpair, 0)

    e_prefetch(0, ib0, si0)
    e_chunk(0, ib0, si0, ib1, si1, first=True)
    e_chunk(1, ib1, si1, ib0, si0, first=False)

    def e_pair(p, carry):
        e_chunk(2 * p, ib0, si0, ib1, si1, first=False)
        e_chunk(2 * p + 1, ib1, si1, ib0, si0, first=False)
        return carry

    lax.fori_loop(1, NCH // 2, e_pair, 0)
    drain_tile(0)
    drain_tile(1)
    e_drain(ib0, si0)

    pltpu.sync_copy(feat6T_hbm, buf.at[pl.ds(0, N * NF6)])
    fs = FT_BASE * wid + jnp.minimum(wid, 4)
    es = fs * TW

    def f_prefetch(c, ib, wb, sem):
        b = es + jnp.minimum(c, FCH - 1) * (FB * TW)
        pltpu.async_copy(rc_hbm.at[pl.ds(b, FB * TW)],
                         ib.at[pl.ds(0, FB * TW)], sem)
        pltpu.async_copy(rc_hbm.at[pl.ds(E + b, FB * TW)],
                         ib.at[pl.ds(FB * TW, FB * TW)], sem)
        pltpu.async_copy(weight_hbm.at[pl.ds(b, FB * TW)], wb, sem)

    def f_drain(ib, wb, sem):
        pltpu.make_async_copy(rc_hbm.at[pl.ds(0, FB * TW)],
                              ib.at[pl.ds(0, FB * TW)], sem).wait()
        pltpu.make_async_copy(rc_hbm.at[pl.ds(0, FB * TW)],
                              ib.at[pl.ds(FB * TW, FB * TW)], sem).wait()
        pltpu.make_async_copy(weight_hbm.at[pl.ds(0, FB * TW)], wb, sem).wait()

    def build_ftile(t, k, ib, wb, slot):
        tile = tiles[slot]
        for g in range(8):
            o = k * TW + g * L
            ni = ib[pl.ds(o, L)]
            nj = ib[pl.ds(FB * TW + o, L)]

            def gcol(nidx, c):
                return plsc.load_gather(buf, [nidx + c * N])

            fa = [gcol(ni, c) for c in range(6)]
            fb = [gcol(nj, c) for c in range(6)]
            dot = fa[0] * fb[0] + fa[1] * fb[1] + fa[2] * fb[2] + fa[3] * fb[3]
            si = fa[0] * fa[0] + fa[1] * fa[1] + fa[2] * fa[2] + fa[3] * fa[3]
            sj = fb[0] * fb[0] + fb[1] * fb[1] + fb[2] * fb[2] + fb[3] * fb[3]
            sim = (dot * _rsqrt(jnp.maximum(si, 1e-16))
                   * _rsqrt(jnp.maximum(sj, 1e-16)))
            dx = fa[4] - fb[4]
            dy = fa[5] - fb[5]
            r = _rsqrt(dx * dx + dy * dy + 1e-12)
            w = jnp.abs(wb[pl.ds(o, L)])
            vals = [w, sim, dx, dy, jnp.abs(dx), jnp.abs(dy), dx * r, dy * r]
            for jj, v in enumerate(vals):
                tile[pl.ds(jj * TW + g * L, L)] = v
        pltpu.async_copy(tile, out_hbm.at[NJT - 1, t], tsems[slot])

    def f_chunk(c, ib, wb, sem, ib_n, wb_n, sem_n, first):
        f_drain(ib, wb, sem)
        f_prefetch(c + 1, ib_n, wb_n, sem_n)
        for k in range(FB):
            if not first:
                drain_tile(k)
            build_ftile(fs + c * FB + k, k, ib, wb, k)

    f_prefetch(0, ib0, wb0, si0)
    f_chunk(0, ib0, wb0, si0, ib1, wb1, si1, first=True)
    f_chunk(1, ib1, wb1, si1, ib0, wb0, si0, first=False)

    def f_pair(p, carry):
        f_chunk(2 * p, ib0, wb0, si0, ib1, wb1, si1, first=False)
        f_chunk(2 * p + 1, ib1, wb1, si1, ib0, wb0, si0, first=False)
        return carry

    lax.fori_loop(1, FCH // 2, f_pair, 0)

    @pl.when(wid < 4)
    def _peel():
        b = es + FT_BASE * TW
        pltpu.sync_copy(rc_hbm.at[pl.ds(b, TW)], ib0.at[pl.ds(0, TW)])
        pltpu.sync_copy(rc_hbm.at[pl.ds(E + b, TW)],
                        ib0.at[pl.ds(FB * TW, TW)])
        pltpu.sync_copy(weight_hbm.at[pl.ds(b, TW)], wb0.at[pl.ds(0, TW)])
        drain_tile(0)
        build_ftile(fs + FT_BASE, 0, ib0, wb0, 0)

    drain_tile(0)
    drain_tile(1)
    drain_tile(2)
    f_drain(ib0, wb0, si0)


@jax.jit
def _encode(embT, rc, edge_weight, feat6T):
    mesh = plsc.VectorSubcoreMesh(core_axis_name="c", subcore_axis_name="s")
    k = pl.kernel(
        _edge_body,
        out_type=jax.ShapeDtypeStruct((NJT, NT, 8 * TW), jnp.float32),
        mesh=mesh,
        scratch_types=[
            pltpu.VMEM((8 * N,), jnp.float32),
            pltpu.VMEM((CE,), jnp.int32),
            pltpu.VMEM((CE,), jnp.int32),
            pltpu.VMEM((FB * TW,), jnp.float32),
            pltpu.VMEM((FB * TW,), jnp.float32),
            pltpu.VMEM((8 * TW,), jnp.float32),
            pltpu.VMEM((8 * TW,), jnp.float32),
            pltpu.VMEM((8 * TW,), jnp.float32),
            pltpu.SemaphoreType.DMA,
            pltpu.SemaphoreType.DMA,
            pltpu.SemaphoreType.DMA,
            pltpu.SemaphoreType.DMA,
            pltpu.SemaphoreType.DMA,
        ],
        compiler_params=pltpu.CompilerParams(needs_layout_passes=False),
    )
    out3 = k(rc, edge_weight, embT, feat6T)
    return (out3.reshape(NJT, NT, 8, TW)
            .transpose(1, 3, 0, 2)
            .reshape(E, OUT_D))


def kernel(node_embeddings, edge_index, edge_weight, node_features):
    embT = node_embeddings.T.reshape(-1)
    rc = jnp.concatenate([edge_index[0], edge_index[1]])
    feat6T = node_features[:, :NF6].T.reshape(-1)
    return _encode(embT, rc, edge_weight, feat6T)


# speedup vs baseline: 1.6642x; 1.6642x over previous
import functools

import jax
import jax.numpy as jnp
from jax import lax
from jax.experimental import pallas as pl
from jax.experimental.pallas import tpu as pltpu
from jax.experimental.pallas import tpu_sc as plsc

N = 10000
E = 320000
H = 128
NF6 = 6
OUT_D = 264

NC = 2
NS = 16
NW = NC * NS
EPW = E // NW
B = 80
NCHUNK = EPW // B
L = 16


def _rsqrt(x):
    xi = lax.bitcast_convert_type(x, jnp.int32)
    yi = jnp.int32(0x5F3759DF) - lax.shift_right_logical(xi, 1)
    y = lax.bitcast_convert_type(yi, jnp.float32)
    xh = x * 0.5
    for _ in range(3):
        y = y * (1.5 - xh * y * y)
    return y


def _edge_body(row_hbm, col_hbm, weight_hbm, emb_hbm, feat6_hbm, out_hbm,
               ir0, ir1, ic0, ic1, wv0, wv1, feat6, pack0, pack1,
               sem_emb0, sem_emb1, sem_out0, sem_out1, sem_i0, sem_i1):
    wid = lax.axis_index("s") * NC + lax.axis_index("c")
    base0 = wid * EPW
    pltpu.sync_copy(feat6_hbm, feat6)

    idx = [(ir0, ic0, wv0, sem_i0), (ir1, ic1, wv1, sem_i1)]
    packs = [(pack0, sem_emb0, sem_out0), (pack1, sem_emb1, sem_out1)]

    def prefetch(g, s):
        ir, ic, wv, sem = idx[s]
        b = jnp.minimum(base0 + g * B, base0 + EPW - B)
        pltpu.async_copy(row_hbm.at[pl.ds(b, B)], ir, sem)
        pltpu.async_copy(col_hbm.at[pl.ds(b, B)], ic, sem)
        pltpu.async_copy(weight_hbm.at[pl.ds(b, B)], wv, sem)

    def drain_prefetch(s):
        ir, ic, wv, sem = idx[s]
        pltpu.make_async_copy(row_hbm.at[pl.ds(0, B)], ir, sem).wait()
        pltpu.make_async_copy(col_hbm.at[pl.ds(0, B)], ic, sem).wait()
        pltpu.make_async_copy(weight_hbm.at[pl.ds(0, B)], wv, sem).wait()

    def gather(s):
        ir, ic, _, _ = idx[s]
        pack, sem_emb, _ = packs[s]
        pltpu.async_copy(emb_hbm.at[ir], pack.at[:, pl.ds(0, H)], sem_emb)
        pltpu.async_copy(emb_hbm.at[ic], pack.at[:, pl.ds(H, H)], sem_emb)

    def wait_gather(s):
        ir, ic, _, _ = idx[s]
        pack, sem_emb, _ = packs[s]
        pltpu.make_async_copy(
            emb_hbm.at[ir], pack.at[:, pl.ds(0, H)], sem_emb).wait()
        pltpu.make_async_copy(
            emb_hbm.at[ic], pack.at[:, pl.ds(H, H)], sem_emb).wait()

    def drain_out(s):
        pack, _, sem_out = packs[s]
        pltpu.make_async_copy(out_hbm.at[pl.ds(0, B)], pack, sem_out).wait()

    def chunk(g, s, drain):
        pack, _, sem_out = packs[s]
        ir, ic, wv, _ = idx[s]
        base = base0 + g * B

        for grp in range(B // L):
            e0 = grp * L
            ni = ir[pl.ds(e0, L)] * NF6
            nj = ic[pl.ds(e0, L)] * NF6

            def gcol(nidx, c):
                return plsc.load_gather(feat6, [nidx + c])

            fa = [gcol(ni, c) for c in range(6)]
            fb = [gcol(nj, c) for c in range(6)]
            dot = fa[0] * fb[0] + fa[1] * fb[1] + fa[2] * fb[2] + fa[3] * fb[3]
            si = fa[0] * fa[0] + fa[1] * fa[1] + fa[2] * fa[2] + fa[3] * fa[3]
            sj = fb[0] * fb[0] + fb[1] * fb[1] + fb[2] * fb[2] + fb[3] * fb[3]
            sim = dot * _rsqrt(jnp.maximum(si, 1e-16)) * _rsqrt(jnp.maximum(sj, 1e-16))
            dx = fa[4] - fb[4]
            dy = fa[5] - fb[5]
            r = _rsqrt(dx * dx + dy * dy + 1e-12)
            w = jnp.abs(wv[pl.ds(e0, L)])
            vals = [w, sim, dx, dy, jnp.abs(dx), jnp.abs(dy), dx * r, dy * r]
            ei = lax.iota(jnp.int32, L) + e0
            for k, v in enumerate(vals):
                kk = jnp.full((L,), 2 * H + k, jnp.int32)
                plsc.store_scatter(pack, [ei, kk], v)

        if drain:
            drain_out(1 - s)
        drain_prefetch(1 - s)
        gather(1 - s)
        wait_gather(s)
        prefetch(g + 2, s)
        pltpu.async_copy(pack, out_hbm.at[pl.ds(base, B)], sem_out)

    prefetch(0, 0)
    drain_prefetch(0)
    gather(0)
    prefetch(1, 1)

    chunk(0, 0, drain=False)

    def pair(p, carry):
        chunk(2 * p + 1, 1, drain=True)
        chunk(2 * p + 2, 0, drain=True)
        return carry

    lax.fori_loop(0, (NCHUNK - 1) // 2, pair, 0)

    drain_out(0)
    wait_gather(1)
    drain_prefetch(0)


@jax.jit
def _encode(node_embeddings, row, col, edge_weight, feat6):
    mesh = plsc.VectorSubcoreMesh(core_axis_name="c", subcore_axis_name="s")
    k = pl.kernel(
        _edge_body,
        out_type=jax.ShapeDtypeStruct((E, OUT_D), jnp.float32),
        mesh=mesh,
        scratch_types=[
            pltpu.VMEM((B,), jnp.int32),
            pltpu.VMEM((B,), jnp.int32),
            pltpu.VMEM((B,), jnp.int32),
            pltpu.VMEM((B,), jnp.int32),
            pltpu.VMEM((B,), jnp.float32),
            pltpu.VMEM((B,), jnp.float32),
            pltpu.VMEM((N * NF6,), jnp.float32),
            pltpu.VMEM((B, OUT_D), jnp.float32),
            pltpu.VMEM((B, OUT_D), jnp.float32),
            pltpu.SemaphoreType.DMA,
            pltpu.SemaphoreType.DMA,
            pltpu.SemaphoreType.DMA,
            pltpu.SemaphoreType.DMA,
            pltpu.SemaphoreType.DMA,
            pltpu.SemaphoreType.DMA,
        ],
        compiler_params=pltpu.CompilerParams(needs_layout_passes=False),
    )
    return k(row, col, edge_weight, node_embeddings, feat6)


def kernel(node_embeddings, edge_index, edge_weight, node_features):
    row = edge_index[0]
    col = edge_index[1]
    feat6 = node_features[:, :NF6].reshape(-1)
    return _encode(node_embeddings, row, col, edge_weight, feat6)
